# baseline (speedup 1.0000x reference)
import functools

import jax
import jax.numpy as jnp
from jax import lax
from jax.experimental import pallas as pl
from jax.experimental.pallas import tpu as pltpu
from jax.experimental.pallas import tpu_sc as plsc

_NC = 2
_NS = 16
_NW = _NC * _NS
_LANES = 16
_CHUNK = 128


def _pool_sc(idx3, emb_table, b_per_w, ctx, n_chunks):
    B = b_per_w * _NW
    _, D = emb_table.shape
    nvr = D // _LANES

    mesh = plsc.VectorSubcoreMesh(
        core_axis_name="c", subcore_axis_name="s",
        num_cores=_NC, num_subcores=_NS)

    @functools.partial(
        pl.kernel,
        out_type=jax.ShapeDtypeStruct((B, D), jnp.float32),
        mesh=mesh,
        scratch_types=[
            pltpu.VMEM((n_chunks, _CHUNK), jnp.int32),
            pltpu.VMEM((n_chunks * _CHUNK, D), jnp.float32),
            pltpu.VMEM((b_per_w, D), jnp.float32),
            pltpu.SemaphoreType.DMA,
        ],
        compiler_params=pltpu.CompilerParams(use_tc_tiling_on_sc=False),
    )
    def pool(idx_hbm, table_hbm, out_hbm, idx_v, rows_v, pooled_v, sem):
        wid = lax.axis_index("s") * _NC + lax.axis_index("c")
        base = wid * b_per_w
        pltpu.sync_copy(idx_hbm.at[wid], idx_v)
        copies = [
            pltpu.async_copy(
                table_hbm.at[idx_v.at[c]],
                rows_v.at[pl.ds(c * _CHUNK, _CHUNK)], sem)
            for c in range(n_chunks)
        ]
        for cp in copies:
            cp.wait()
        scale = jnp.float32(1.0 / ctx)
        for i in range(b_per_w):
            def body(j, accs, i=i):
                r = i * ctx + j
                return tuple(
                    a + rows_v[r, pl.ds(v * _LANES, _LANES)]
                    for v, a in enumerate(accs))
            accs = lax.fori_loop(
                0, ctx, body,
                tuple(jnp.zeros((_LANES,), jnp.float32) for _ in range(nvr)))
            for v in range(nvr):
                pooled_v[i, pl.ds(v * _LANES, _LANES)] = accs[v] * scale
        pltpu.sync_copy(pooled_v, out_hbm.at[pl.ds(base, b_per_w)])

    return pool(idx3, emb_table)


def _transpose_tc(w, v_tile=4096):
    V, D = w.shape
    n = -(-V // v_tile)

    def tr(w_ref, o_ref):
        o_ref[...] = w_ref[...].T

    return pl.pallas_call(
        tr,
        grid=(n,),
        in_specs=[pl.BlockSpec((v_tile, D), lambda i: (i, 0))],
        out_specs=pl.BlockSpec((D, v_tile), lambda i: (0, i)),
        out_shape=jax.ShapeDtypeStruct((D, V), jnp.float32),
    )(w)


def _project_tc(pooled, lin_w, lin_b, rows):
    B, D = pooled.shape
    V = lin_w.shape[0]
    n_tiles = B // rows
    b2 = lin_b.reshape(1, V)
    wt = _transpose_tc(lin_w)

    def mm(p_ref, w_ref, b_ref, o_ref):
        o_ref[...] = lax.dot_general(
            p_ref[...], w_ref[...],
            dimension_numbers=(((1,), (0,)), ((), ())),
            preferred_element_type=jnp.float32) + b_ref[...]

    return pl.pallas_call(
        mm,
        grid=(n_tiles,),
        in_specs=[
            pl.BlockSpec((rows, D), lambda i: (i, 0)),
            pl.BlockSpec((D, V), lambda i: (0, 0)),
            pl.BlockSpec((1, V), lambda i: (0, 0)),
        ],
        out_specs=pl.BlockSpec((rows, V), lambda i: (i, 0)),
        out_shape=jax.ShapeDtypeStruct((B, V), jnp.float32),
    )(pooled, wt, b2)


def kernel(context_words, emb_table, lin_w, lin_b):
    B, ctx = context_words.shape
    b_per_w = B // _NW
    per_w = b_per_w * ctx
    n_chunks = -(-per_w // _CHUNK)
    pad = n_chunks * _CHUNK - per_w
    idx = context_words.astype(jnp.int32).reshape(_NW, per_w)
    if pad:
        idx = jnp.pad(idx, ((0, 0), (0, pad)))
    idx3 = idx.reshape(_NW, n_chunks, _CHUNK)
    pooled = _pool_sc(idx3, emb_table, b_per_w, ctx, n_chunks)
    return _project_tc(pooled, lin_w, lin_b, 16)


# speedup vs baseline: 1.6371x; 1.6371x over previous
import functools

import jax
import jax.numpy as jnp
from jax import lax
from jax.experimental import pallas as pl
from jax.experimental.pallas import tpu as pltpu
from jax.experimental.pallas import tpu_sc as plsc

_NC = 2
_NS = 16
_NW = _NC * _NS
_LANES = 16
_CHUNK = 128


def _pool_sc(idx3, emb_table, lin_w, b_per_w, ctx, n_chunks):
    B = b_per_w * _NW
    V, D = emb_table.shape
    nvr = D // _LANES
    wrows = 1000
    n_wchunks = -(-V // wrows)
    w_iters = -(-n_wchunks // _NW)

    mesh = plsc.VectorSubcoreMesh(
        core_axis_name="c", subcore_axis_name="s",
        num_cores=_NC, num_subcores=_NS)

    @functools.partial(
        pl.kernel,
        out_type=(jax.ShapeDtypeStruct((B, D), jnp.float32),
                  jax.ShapeDtypeStruct((V, D), jnp.float32)),
        mesh=mesh,
        scratch_types=[
            pltpu.VMEM((n_chunks, _CHUNK), jnp.int32),
            pltpu.VMEM((n_chunks * _CHUNK, D), jnp.float32),
            pltpu.VMEM((b_per_w, D), jnp.float32),
            pltpu.VMEM((2, wrows, D), jnp.float32),
            pltpu.SemaphoreType.DMA,
            pltpu.SemaphoreType.DMA((2,)),
        ],
        compiler_params=pltpu.CompilerParams(use_tc_tiling_on_sc=False),
    )
    def pool(idx_hbm, table_hbm, w_hbm, out_hbm, wout_hbm,
             idx_v, rows_v, pooled_v, wbuf_v, sem, wsems):
        wid = lax.axis_index("s") * _NC + lax.axis_index("c")
        base = wid * b_per_w
        pltpu.sync_copy(idx_hbm.at[wid], idx_v)
        copies = [
            pltpu.async_copy(
                table_hbm.at[idx_v.at[c]],
                rows_v.at[pl.ds(c * _CHUNK, _CHUNK)], sem)
            for c in range(n_chunks)
        ]
        for cp in copies:
            cp.wait()
        scale = jnp.float32(1.0 / ctx)
        for i in range(b_per_w):
            def body(j, accs, i=i):
                r = i * ctx + j
                return tuple(
                    a + rows_v[r, pl.ds(v * _LANES, _LANES)]
                    for v, a in enumerate(accs))
            accs = lax.fori_loop(
                0, ctx, body,
                tuple(jnp.zeros((_LANES,), jnp.float32) for _ in range(nvr)))
            for v in range(nvr):
                pooled_v[i, pl.ds(v * _LANES, _LANES)] = accs[v] * scale
        pltpu.sync_copy(pooled_v, out_hbm.at[pl.ds(base, b_per_w)])
        for it in range(w_iters):
            chunk = wid + _NW * it
            row0 = chunk * wrows

            @pl.when(chunk < n_wchunks)
            def _(it=it, row0=row0):
                buf = it % 2
                if it >= 2:
                    pltpu.make_async_copy(
                        wbuf_v.at[buf], wout_hbm.at[pl.ds(0, wrows)],
                        wsems.at[buf]).wait()
                pltpu.async_copy(
                    w_hbm.at[pl.ds(row0, wrows)], wbuf_v.at[buf],
                    wsems.at[buf]).wait()
                pltpu.async_copy(
                    wbuf_v.at[buf], wout_hbm.at[pl.ds(row0, wrows)],
                    wsems.at[buf]).start()
        for b in range(2):
            pltpu.make_async_copy(
                wbuf_v.at[b], wout_hbm.at[pl.ds(0, wrows)],
                wsems.at[b]).wait()

    return pool(idx3, emb_table, lin_w)


def _transpose_tc(w, v_tile=4096):
    V, D = w.shape
    n = -(-V // v_tile)

    def tr(w_ref, o_ref):
        o_ref[...] = w_ref[...].T

    return pl.pallas_call(
        tr,
        grid=(n,),
        in_specs=[pl.BlockSpec((v_tile, D), lambda i: (i, 0))],
        out_specs=pl.BlockSpec((D, v_tile), lambda i: (0, i)),
        out_shape=jax.ShapeDtypeStruct((D, V), jnp.float32),
    )(w)


def _project_tc(pooled, lin_w, lin_b, rows):
    B, D = pooled.shape
    V = lin_w.shape[0]
    n_tiles = B // rows
    b2 = lin_b.reshape(1, V)
    wt = _transpose_tc(lin_w)

    def mm(p_ref, w_ref, b_ref, o_ref):
        o_ref[...] = lax.dot_general(
            p_ref[...], w_ref[...],
            dimension_numbers=(((1,), (0,)), ((), ())),
            preferred_element_type=jnp.float32) + b_ref[...]

    return pl.pallas_call(
        mm,
        grid=(n_tiles,),
        in_specs=[
            pl.BlockSpec((rows, D), lambda i: (i, 0)),
            pl.BlockSpec((D, V), lambda i: (0, 0)),
            pl.BlockSpec((1, V), lambda i: (0, 0)),
        ],
        out_specs=pl.BlockSpec((rows, V), lambda i: (i, 0)),
        out_shape=jax.ShapeDtypeStruct((B, V), jnp.float32),
    )(pooled, wt, b2)


def kernel(context_words, emb_table, lin_w, lin_b):
    B, ctx = context_words.shape
    b_per_w = B // _NW
    per_w = b_per_w * ctx
    n_chunks = -(-per_w // _CHUNK)
    pad = n_chunks * _CHUNK - per_w
    idx = context_words.astype(jnp.int32).reshape(_NW, per_w)
    if pad:
        idx = jnp.pad(idx, ((0, 0), (0, pad)))
    idx3 = idx.reshape(_NW, n_chunks, _CHUNK)
    pooled, w_std = _pool_sc(idx3, emb_table, lin_w, b_per_w, ctx, n_chunks)
    return _project_tc(pooled, w_std, lin_b, 16)
